# Initial kernel scaffold; baseline (speedup 1.0000x reference)
import functools

import jax
import jax.numpy as jnp
from jax import lax
from jax.experimental import pallas as pl
from jax.experimental.pallas import tpu as pltpu
from jax.experimental.pallas import tpu_sc as plsc

VOCAB = 1048576
EMBED_DIM = 32
BATCH = 16384
N_FIELDS = 26
TOT = BATCH * N_FIELDS
NUM_WORKERS = 32
PER_W = TOT // NUM_WORKERS
N_CHUNK = 8
CHUNK = PER_W // N_CHUNK


def _make_gather():
    mesh = plsc.VectorSubcoreMesh(core_axis_name="c", subcore_axis_name="s")

    @functools.partial(
        pl.kernel,
        mesh=mesh,
        out_type=jax.ShapeDtypeStruct((TOT, EMBED_DIM), jnp.float32),
        scratch_types=[
            pltpu.VMEM((CHUNK,), jnp.int32),
            pltpu.VMEM((CHUNK, EMBED_DIM), jnp.float32),
            pltpu.SemaphoreType.DMA,
        ],
    )
    def gather_kernel(ids_hbm, table_hbm, out_hbm, idx_v, rows_v, sem):
        wid = lax.axis_index("s") * 2 + lax.axis_index("c")
        base = wid * PER_W

        def body(i, carry):
            off = base + i * CHUNK
            pltpu.sync_copy(ids_hbm.at[pl.ds(off, CHUNK)], idx_v)
            pltpu.async_copy(table_hbm.at[idx_v], rows_v, sem).wait()
            pltpu.sync_copy(rows_v, out_hbm.at[pl.ds(off, CHUNK)])
            return carry

        lax.fori_loop(0, N_CHUNK, body, 0)

    return gather_kernel


_gather = _make_gather()


@jax.jit
def kernel(ids, table):
    flat = ids.reshape(TOT).astype(jnp.int32)
    out = _gather(flat, table)
    return out.reshape(BATCH, N_FIELDS, EMBED_DIM)


# speedup vs baseline: 1.6470x; 1.6470x over previous
import functools

import jax
import jax.numpy as jnp
from jax import lax
from jax.experimental import pallas as pl
from jax.experimental.pallas import tpu as pltpu
from jax.experimental.pallas import tpu_sc as plsc

VOCAB = 1048576
EMBED_DIM = 32
BATCH = 16384
N_FIELDS = 26
TOT = BATCH * N_FIELDS
NUM_WORKERS = 32
PER_W = TOT // NUM_WORKERS
N_CHUNK = 8
CHUNK = PER_W // N_CHUNK


def _make_gather():
    mesh = plsc.VectorSubcoreMesh(core_axis_name="c", subcore_axis_name="s")

    @functools.partial(
        pl.kernel,
        mesh=mesh,
        out_type=jax.ShapeDtypeStruct((TOT, EMBED_DIM), jnp.float32),
        scratch_types=[
            pltpu.VMEM((CHUNK,), jnp.int32),
            pltpu.VMEM((CHUNK, EMBED_DIM), jnp.float32),
            pltpu.SemaphoreType.DMA,
        ],
        compiler_params=pltpu.CompilerParams(use_tc_tiling_on_sc=False),
    )
    def gather_kernel(ids_hbm, table_hbm, out_hbm, idx_v, rows_v, sem):
        wid = lax.axis_index("s") * 2 + lax.axis_index("c")
        base = wid * PER_W

        def body(i, carry):
            off = base + i * CHUNK
            pltpu.sync_copy(ids_hbm.at[pl.ds(off, CHUNK)], idx_v)
            pltpu.async_copy(table_hbm.at[idx_v], rows_v, sem).wait()
            pltpu.sync_copy(rows_v, out_hbm.at[pl.ds(off, CHUNK)])
            return carry

        lax.fori_loop(0, N_CHUNK, body, 0)

    return gather_kernel


_gather = _make_gather()


@jax.jit
def kernel(ids, table):
    flat = ids.reshape(TOT).astype(jnp.int32)
    out = _gather(flat, table)
    return out.reshape(BATCH, N_FIELDS, EMBED_DIM)


# speedup vs baseline: 1.6500x; 1.0018x over previous
import functools

import jax
import jax.numpy as jnp
from jax import lax
from jax.experimental import pallas as pl
from jax.experimental.pallas import tpu as pltpu
from jax.experimental.pallas import tpu_sc as plsc

VOCAB = 1048576
EMBED_DIM = 32
BATCH = 16384
N_FIELDS = 26
TOT = BATCH * N_FIELDS
NUM_WORKERS = 32
PER_W = TOT // NUM_WORKERS
N_CHUNK = 8
CHUNK = PER_W // N_CHUNK


def _make_gather():
    mesh = plsc.VectorSubcoreMesh(core_axis_name="c", subcore_axis_name="s")
    NBUF = 2

    @functools.partial(
        pl.kernel,
        mesh=mesh,
        out_type=jax.ShapeDtypeStruct((TOT, EMBED_DIM), jnp.float32),
        scratch_types=[
            pltpu.VMEM((NBUF, CHUNK), jnp.int32),
            pltpu.VMEM((NBUF, CHUNK, EMBED_DIM), jnp.float32),
            [pltpu.SemaphoreType.DMA] * NBUF,
            [pltpu.SemaphoreType.DMA] * NBUF,
            [pltpu.SemaphoreType.DMA] * NBUF,
        ],
        compiler_params=pltpu.CompilerParams(use_tc_tiling_on_sc=False),
    )
    def gather_kernel(ids_hbm, table_hbm, out_hbm, idx_v, rows_v,
                      sem_i, sem_g, sem_o):
        wid = lax.axis_index("s") * 2 + lax.axis_index("c")
        base = wid * PER_W

        def idx_copy(i, p):
            return pltpu.async_copy(
                ids_hbm.at[pl.ds(base + i * CHUNK, CHUNK)],
                idx_v.at[p], sem_i[p])

        out_handles = [None] * NBUF
        idx_copy(0, 0)
        for i in range(N_CHUNK):
            p = i % NBUF
            if i + 1 < N_CHUNK:
                idx_copy(i + 1, (i + 1) % NBUF)
            pltpu.make_async_copy(
                ids_hbm.at[pl.ds(base + i * CHUNK, CHUNK)],
                idx_v.at[p], sem_i[p]).wait()
            if out_handles[p] is not None:
                out_handles[p].wait()
            gather = pltpu.async_copy(
                table_hbm.at[idx_v.at[p]], rows_v.at[p], sem_g[p])
            gather.wait()
            out_handles[p] = pltpu.async_copy(
                rows_v.at[p], out_hbm.at[pl.ds(base + i * CHUNK, CHUNK)],
                sem_o[p])
        for h in out_handles:
            if h is not None:
                h.wait()

    return gather_kernel


_gather = _make_gather()


@jax.jit
def kernel(ids, table):
    flat = ids.reshape(TOT).astype(jnp.int32)
    out = _gather(flat, table)
    return out.reshape(BATCH, N_FIELDS, EMBED_DIM)


# speedup vs baseline: 1.6577x; 1.0046x over previous
import functools

import jax
import jax.numpy as jnp
from jax import lax
from jax.experimental import pallas as pl
from jax.experimental.pallas import tpu as pltpu
from jax.experimental.pallas import tpu_sc as plsc

VOCAB = 1048576
EMBED_DIM = 32
BATCH = 16384
N_FIELDS = 26
TOT = BATCH * N_FIELDS
NUM_WORKERS = 32
PER_W = TOT // NUM_WORKERS
N_CHUNK = 16
CHUNK = PER_W // N_CHUNK
NBUF = 4
LAG = 3


def _make_gather():
    mesh = plsc.VectorSubcoreMesh(core_axis_name="c", subcore_axis_name="s")

    @functools.partial(
        pl.kernel,
        mesh=mesh,
        out_type=jax.ShapeDtypeStruct((TOT, EMBED_DIM), jnp.float32),
        scratch_types=[
            pltpu.VMEM((NBUF, CHUNK), jnp.int32),
            pltpu.VMEM((NBUF, CHUNK, EMBED_DIM), jnp.float32),
            [pltpu.SemaphoreType.DMA] * NBUF,
            [pltpu.SemaphoreType.DMA] * NBUF,
            [pltpu.SemaphoreType.DMA] * NBUF,
        ],
        compiler_params=pltpu.CompilerParams(use_tc_tiling_on_sc=False),
    )
    def gather_kernel(ids_hbm, table_hbm, out_hbm, idx_v, rows_v,
                      sem_i, sem_g, sem_o):
        wid = lax.axis_index("s") * 2 + lax.axis_index("c")
        base = wid * PER_W

        def fire_idx(i):
            p = i % NBUF
            return pltpu.async_copy(
                ids_hbm.at[pl.ds(base + i * CHUNK, CHUNK)],
                idx_v.at[p], sem_i[p])

        idx_handles = {}
        gather_handles = {}
        out_handles = [None] * NBUF
        for i in range(min(NBUF, N_CHUNK)):
            idx_handles[i] = fire_idx(i)

        for i in range(N_CHUNK + LAG):
            if i < N_CHUNK:
                p = i % NBUF
                idx_handles[i].wait()
                if out_handles[p] is not None:
                    out_handles[p].wait()
                    out_handles[p] = None
                gather_handles[i] = pltpu.async_copy(
                    table_hbm.at[idx_v.at[p]], rows_v.at[p], sem_g[p])
            j = i - LAG
            if 0 <= j < N_CHUNK:
                q = j % NBUF
                gather_handles[j].wait()
                if j + NBUF < N_CHUNK:
                    idx_handles[j + NBUF] = fire_idx(j + NBUF)
                out_handles[q] = pltpu.async_copy(
                    rows_v.at[q], out_hbm.at[pl.ds(base + j * CHUNK, CHUNK)],
                    sem_o[q])
        for h in out_handles:
            if h is not None:
                h.wait()

    return gather_kernel


_gather = _make_gather()


@jax.jit
def kernel(ids, table):
    flat = ids.reshape(TOT).astype(jnp.int32)
    out = _gather(flat, table)
    return out.reshape(BATCH, N_FIELDS, EMBED_DIM)
